# baseline (speedup 1.0000x reference)
import functools

import jax
import jax.numpy as jnp
from jax import lax
from jax.experimental import pallas as pl
from jax.experimental.pallas import tpu as pltpu
from jax.experimental.pallas import tpu_sc as plsc

_P1P2 = 10000
_P2 = 100
_B = 4096
_D = 128
_NC = 2
_NS = 16
_NW = _NC * _NS
_BW = _B // _NW
_CH = 8
_NCHUNK = _BW // _CH


def _sc_call(idx, c0t, c1, c2rep):
    mesh = plsc.VectorSubcoreMesh(core_axis_name="c", subcore_axis_name="s")

    @functools.partial(
        pl.kernel,
        out_type=jax.ShapeDtypeStruct((_B, _D), jnp.float32),
        mesh=mesh,
        compiler_params=pltpu.CompilerParams(needs_layout_passes=False),
        scratch_types=[
            pltpu.VMEM((_BW,), jnp.int32),
            pltpu.VMEM((_BW,), jnp.int32),
            pltpu.VMEM((_BW,), jnp.int32),
            pltpu.VMEM((_BW,), jnp.int32),
            pltpu.VMEM((_CH, 512), jnp.float32),
            pltpu.VMEM((_CH, 8192), jnp.float32),
            pltpu.VMEM((_CH, 1024), jnp.float32),
            pltpu.VMEM((1024,), jnp.float32),
            pltpu.VMEM((_BW, _D), jnp.float32),
            pltpu.SemaphoreType.DMA,
            pltpu.SemaphoreType.DMA,
            pltpu.SemaphoreType.DMA,
        ],
    )
    def k(idx_hbm, c0t_hbm, c1_hbm, c2rep_hbm, out_hbm,
          idx_v, i0_v, i1_v, i2_v, g0_v, c1_v, g2_v, t_v, out_v,
          sem0, sem1, sem2):
        wid = lax.axis_index("s") * _NC + lax.axis_index("c")
        base = wid * _BW
        pltpu.sync_copy(idx_hbm.at[pl.ds(base, _BW)], idx_v)

        for kk in range(_BW // 16):
            sl = pl.ds(kk * 16, 16)
            v = idx_v[sl]
            q = v // _P2
            i2_v[sl] = v - q * _P2
            i0 = q // _P2
            i0_v[sl] = i0
            i1_v[sl] = q - i0 * _P2

        lane = lax.iota(jnp.int32, 16)
        t_base = [
            [(a * 256 + (h * 4 + lane // 4) * 32) for h in range(2)]
            for a in range(4)
        ]

        def chunk_body(g, _):
            cbase = g * _CH
            cp0 = pltpu.async_copy(c0t_hbm.at[i0_v.at[pl.ds(cbase, _CH)]],
                                   g0_v, sem0)
            cp1 = pltpu.async_copy(c1_hbm.at[i1_v.at[pl.ds(cbase, _CH)]],
                                   c1_v, sem1)
            cp2 = pltpu.async_copy(c2rep_hbm.at[i2_v.at[pl.ds(cbase, _CH)]],
                                   g2_v, sem2)
            cp0.wait()
            cp1.wait()
            cp2.wait()

            def item_body(j, _):
                for h in range(2):
                    acc = [[jnp.zeros((16,), jnp.float32) for _ in range(8)]
                           for _ in range(4)]

                    def r_body(r, acc_flat):
                        acc_ = [list(acc_flat[a * 8:(a + 1) * 8])
                                for a in range(4)]
                        av = g0_v[j, pl.ds(r * 16, 16)]
                        cv = [c1_v[j, pl.ds(r * 256 + h * 128 + mm * 16, 16)]
                              for mm in range(8)]
                        for a in range(4):
                            s_a = av[a]
                            for mm in range(8):
                                acc_[a][mm] = acc_[a][mm] + s_a * cv[mm]
                        return tuple(x for row in acc_ for x in row)

                    acc_flat = lax.fori_loop(
                        0, 32, r_body,
                        tuple(x for row in acc for x in row))
                    for a in range(4):
                        for mm in range(8):
                            t_v[pl.ds(a * 256 + h * 128 + mm * 16, 16)] = (
                                acc_flat[a * 8 + mm])

                def s_body(s, o_flat):
                    o = list(o_flat)
                    g2v = [g2_v[j, pl.ds(s * 32 + h * 16, 16)]
                           for h in range(2)]
                    for a in range(4):
                        for h in range(2):
                            tv = plsc.load_gather(t_v, [t_base[a][h] + s])
                            o[a * 2 + h] = o[a * 2 + h] + tv * g2v[h]
                    return tuple(o)

                o_flat = lax.fori_loop(
                    0, 32, s_body,
                    tuple(jnp.zeros((16,), jnp.float32) for _ in range(8)))
                for a in range(4):
                    for h in range(2):
                        out_v[cbase + j, pl.ds(a * 32 + h * 16, 16)] = (
                            o_flat[a * 2 + h])
                return 0

            lax.fori_loop(0, _CH, item_body, 0)
            return 0

        lax.fori_loop(0, _NCHUNK, chunk_body, 0)
        pltpu.sync_copy(out_v, out_hbm.at[pl.ds(base, _BW)])

    return k(idx, c0t, c1, c2rep)


@jax.jit
def kernel(indices, core0, core1, core2):
    idx = indices.astype(jnp.int32)
    c0t = jnp.pad(
        jnp.transpose(core0.reshape(100, 4, 32), (0, 2, 1)),
        ((0, 0), (0, 0), (0, 12)),
    ).reshape(100, 512)
    c2rep = jnp.tile(core2.reshape(100, 32, 1, 4), (1, 1, 8, 1)).reshape(100, 1024)
    return _sc_call(idx, c0t, core1, c2rep)


# speedup vs baseline: 2.1518x; 2.1518x over previous
import functools

import jax
import jax.numpy as jnp
from jax import lax
from jax.experimental import pallas as pl
from jax.experimental.pallas import tpu as pltpu
from jax.experimental.pallas import tpu_sc as plsc

_P2 = 100
_B = 4096
_D = 128
_NC = 2
_NS = 16
_NW = _NC * _NS
_BW = _B // _NW
_CH = 4
_NCHUNK = _BW // _CH

_ARG_SPECS = (
    jax.ShapeDtypeStruct((_B,), jnp.int32),
    jax.ShapeDtypeStruct((100, 144), jnp.float32),
    jax.ShapeDtypeStruct((100, 8192), jnp.float32),
    jax.ShapeDtypeStruct((100, 128), jnp.float32),
)


def _sc_call(idx, c0t, c1, c2):
    mesh = plsc.VectorSubcoreMesh(core_axis_name="c", subcore_axis_name="s")

    @functools.partial(
        pl.kernel,
        out_type=jax.ShapeDtypeStruct((_B, _D), jnp.float32),
        mesh=mesh,
        compiler_params=pltpu.CompilerParams(
            needs_layout_passes=False, use_tc_tiling_on_sc=False),
        scratch_types=[
            pltpu.VMEM((_BW,), jnp.int32),
            pltpu.VMEM((_NCHUNK, 8), jnp.int32),
            pltpu.VMEM((_NCHUNK, 8), jnp.int32),
            pltpu.VMEM((_NCHUNK, 8), jnp.int32),
            pltpu.VMEM((_BW, 144), jnp.float32),
            pltpu.VMEM((2, _CH, 8192), jnp.float32),
            pltpu.VMEM((_BW, _D), jnp.float32),
            pltpu.VMEM((1024,), jnp.float32),
            pltpu.VMEM((_BW, _D), jnp.float32),
            pltpu.SemaphoreType.DMA,
            pltpu.SemaphoreType.DMA,
            pltpu.SemaphoreType.DMA,
        ],
    )
    def k(idx_hbm, c0t_hbm, c1_hbm, c2_hbm, out_hbm,
          idx_v, i0c_v, i1c_v, i2c_v, g0_v, c1_v, g2_v, t_v, out_v,
          semg, sem0, sem1):
        wid = lax.axis_index("s") * _NC + lax.axis_index("c")
        base = wid * _BW
        pltpu.sync_copy(idx_hbm.at[pl.ds(base, _BW)], idx_v)

        lane = lax.iota(jnp.int32, 16)
        for kk in range(_BW // 16):
            sl = pl.ds(kk * 16, 16)
            v = idx_v[sl]
            q = v // _P2
            i2d = v - q * _P2
            i0d = q // _P2
            i1d = q - i0d * _P2
            rows = kk * 4 + lane // 4
            cols = lane % 4
            plsc.store_scatter(i0c_v, [rows, cols], i0d)
            plsc.store_scatter(i1c_v, [rows, cols], i1d)
            plsc.store_scatter(i2c_v, [rows, cols], i2d)

        def flat_digits(dst_v, which):
            for kk in range(_BW // 16):
                sl = pl.ds(kk * 16, 16)
                v = idx_v[sl]
                q = v // _P2
                if which == 0:
                    dst_v[sl] = q // _P2
                else:
                    dst_v[sl] = v - q * _P2

        def body_with_scratch(i0f_v, i2f_v):
            flat_digits(i0f_v, 0)
            flat_digits(i2f_v, 2)
            cp0 = pltpu.async_copy(c0t_hbm.at[i0f_v], g0_v, sem0)
            cp2 = pltpu.async_copy(c2_hbm.at[i2f_v], g2_v, sem1)
            cp0.wait()
            cp2.wait()

        pl.run_scoped(body_with_scratch,
                      pltpu.VMEM((_BW,), jnp.int32),
                      pltpu.VMEM((_BW,), jnp.int32))

        t_base = [
            [(a * 256 + h * 4 + lane // 4) for h in range(2)]
            for a in range(4)
        ]
        dpat = lane % 4
        sems = [sem0, sem1]

        def start_fetch(g, ph):
            gc = jnp.minimum(g, _NCHUNK - 1)
            pltpu.async_copy(c1_hbm.at[i1c_v.at[gc, pl.ds(0, _CH)]],
                             c1_v.at[ph], sems[ph])

        def wait_fetch(ph):
            pltpu.make_async_copy(c1_hbm.at[pl.ds(0, _CH)], c1_v.at[ph],
                                  sems[ph]).wait()

        def compute_chunk(g, ph):
            def item_body(j, _):
                b = g * _CH + j
                for h in range(2):
                    def r_body(r, acc_flat):
                        acc_ = list(acc_flat)
                        av = g0_v[b, pl.ds(r * 4, 16)]
                        cv = [c1_v[ph, j, pl.ds(r * 256 + h * 128 + mm * 16, 16)]
                              for mm in range(8)]
                        for a in range(4):
                            s_a = av[a]
                            for mm in range(8):
                                acc_[a * 8 + mm] = acc_[a * 8 + mm] + s_a * cv[mm]
                        return tuple(acc_)

                    acc_flat = lax.fori_loop(
                        0, 32, r_body,
                        tuple(jnp.zeros((16,), jnp.float32) for _ in range(32)),
                        unroll=2)
                    for a in range(4):
                        for mm in range(8):
                            t_v[pl.ds(a * 256 + h * 128 + mm * 16, 16)] = (
                                acc_flat[a * 8 + mm])

                rowv = lane * 0 + b

                def s_body(s, o_flat):
                    o = list(o_flat)
                    g2v = plsc.load_gather(g2_v, [rowv, s * 4 + dpat])
                    for a in range(4):
                        for h in range(2):
                            tv = plsc.load_gather(t_v, [t_base[a][h] + s * 8])
                            o[a * 2 + h] = o[a * 2 + h] + tv * g2v
                    return tuple(o)

                o_flat = lax.fori_loop(
                    0, 32, s_body,
                    tuple(jnp.zeros((16,), jnp.float32) for _ in range(8)),
                    unroll=2)
                for a in range(4):
                    for h in range(2):
                        out_v[b, pl.ds(a * 32 + h * 16, 16)] = (
                            o_flat[a * 2 + h])
                return 0

            lax.fori_loop(0, _CH, item_body, 0)

        start_fetch(0, 0)

        def ring_body(gg, _):
            g = gg * 2
            start_fetch(g + 1, 1)
            wait_fetch(0)
            compute_chunk(g, 0)
            start_fetch(g + 2, 0)
            wait_fetch(1)
            compute_chunk(g + 1, 1)
            return 0

        lax.fori_loop(0, _NCHUNK // 2, ring_body, 0)
        wait_fetch(0)
        pltpu.sync_copy(out_v, out_hbm.at[pl.ds(base, _BW)])

    return k(idx, c0t, c1, c2)


def _prep(indices, core0, core1, core2):
    idx = indices.astype(jnp.int32)
    c0t = jnp.pad(
        jnp.transpose(core0.reshape(100, 4, 32), (0, 2, 1)).reshape(100, 128),
        ((0, 0), (0, 16)),
    )
    c1t = jnp.transpose(core1.reshape(100, 32, 8, 32), (0, 1, 3, 2))
    c1t = c1t.reshape(100, 8192)
    return idx, c0t, c1t, core2


@jax.jit
def kernel(indices, core0, core1, core2):
    return _sc_call(*_prep(indices, core0, core1, core2))
